# baseline (speedup 1.0000x reference)
import functools

import jax
import jax.numpy as jnp
from jax import lax
from jax.experimental import pallas as pl
from jax.experimental.pallas import tpu as pltpu
from jax.experimental.pallas import tpu_sc as plsc

N_LEVELS = 256
VMIN, VMAX = -4.0, 4.0
Q_STEP = (VMAX - VMIN) / (N_LEVELS - 1)
INV_W = 1.0 / Q_STEP
BIAS = -(VMIN - Q_STEP / 2.0) * INV_W + 1.0

LANES = 16
NW = 32
NBINS = 258
TW = NBINS * LANES
TOTAL = 1 * 16 * 2048 * 2048
PER_TILE = TOTAL // NW
CHUNK = 32768
NCHUNK = PER_TILE // CHUNK
NPAIR = NCHUNK // 2
UNROLL = 4
SCALE = 1.0 / TOTAL

_mesh = plsc.VectorSubcoreMesh(core_axis_name="c", subcore_axis_name="s")


@functools.partial(
    pl.kernel,
    mesh=_mesh,
    out_type=jax.ShapeDtypeStruct((NW, TW), jnp.float32),
    scratch_types=[
        pltpu.VMEM((CHUNK,), jnp.float32),
        pltpu.VMEM((CHUNK,), jnp.float32),
        pltpu.VMEM((TW,), jnp.float32),
        pltpu.SemaphoreType.DMA,
        pltpu.SemaphoreType.DMA,
    ],
    compiler_params=pltpu.CompilerParams(needs_layout_passes=False),
)
def _hist_sc(x_hbm, out_hbm, buf0, buf1, table, sem0, sem1):
    wid = lax.axis_index("s") * 2 + lax.axis_index("c")
    base = wid * PER_TILE

    zeros16 = jnp.zeros((LANES,), jnp.float32)

    def zero_body(i, c):
        table[pl.ds(i * LANES, LANES)] = zeros16
        return c

    lax.fori_loop(0, TW // LANES, zero_body, 0)

    lane = lax.iota(jnp.int32, LANES)
    ones16 = jnp.ones((LANES,), jnp.float32)

    def process(buf):
        def body(i, c):
            for u in range(UNROLL):
                v = buf[pl.ds((i * UNROLL + u) * LANES, LANES)]
                t = v * INV_W + BIAS
                t = jnp.minimum(jnp.maximum(t, 0.0), float(N_LEVELS + 1))
                idx = lax.shift_left(t.astype(jnp.int32), 4) + lane
                plsc.addupdate_scatter(table, [idx], ones16)
            return c

        lax.fori_loop(0, CHUNK // (LANES * UNROLL), body, 0)

    def start(g, buf, sem):
        off = pl.multiple_of(base + g * CHUNK, CHUNK)
        return pltpu.async_copy(x_hbm.at[pl.ds(off, CHUNK)], buf, sem)

    def wait(buf, sem):
        pltpu.make_async_copy(x_hbm.at[pl.ds(base, CHUNK)], buf, sem).wait()

    start(0, buf0, sem0)
    start(1, buf1, sem1)

    def pair_body(p, c):
        g = p * 2
        wait(buf0, sem0)
        process(buf0)
        start(g + 2, buf0, sem0)
        wait(buf1, sem1)
        process(buf1)
        start(g + 3, buf1, sem1)
        return c

    lax.fori_loop(0, NPAIR - 1, pair_body, 0)

    wait(buf0, sem0)
    process(buf0)
    wait(buf1, sem1)
    process(buf1)

    def scale_body(i, c):
        table[pl.ds(i * LANES, LANES)] = table[pl.ds(i * LANES, LANES)] * SCALE
        return c

    lax.fori_loop(0, TW // LANES, scale_body, 0)

    pltpu.sync_copy(table, out_hbm.at[wid])


def kernel(x, bins_edges):
    parts = _hist_sc(x.reshape(TOTAL))
    density = jnp.sum(parts.reshape(NW, NBINS, LANES), axis=(0, 2))[1 : N_LEVELS + 1]
    return (x, density)


# speedup vs baseline: 3.3341x; 3.3341x over previous
import functools

import jax
import jax.numpy as jnp
from jax import lax
from jax.experimental import pallas as pl
from jax.experimental.pallas import tpu as pltpu
from jax.experimental.pallas import tpu_sc as plsc

N_LEVELS = 256
VMIN, VMAX = -4.0, 4.0
Q_STEP = (VMAX - VMIN) / (N_LEVELS - 1)
INV_W = 1.0 / Q_STEP
BIAS = -(VMIN - Q_STEP / 2.0) * INV_W + 1.0

LANES = 16
NW = 32
NBINS = 258
TW = NBINS * LANES
TOTAL = 1 * 16 * 2048 * 2048
PER_TILE = TOTAL // NW
CHUNK = 32768
NCHUNK = PER_TILE // CHUNK
NPAIR = NCHUNK // 2
UNROLL = 4
SCALE = 1.0 / TOTAL

_mesh = plsc.VectorSubcoreMesh(core_axis_name="c", subcore_axis_name="s")


@functools.partial(
    pl.kernel,
    mesh=_mesh,
    out_type=jax.ShapeDtypeStruct((NW, TW), jnp.float32),
    scratch_types=[
        pltpu.VMEM((CHUNK,), jnp.float32),
        pltpu.VMEM((CHUNK,), jnp.float32),
        pltpu.VMEM((TW,), jnp.float32),
        pltpu.SemaphoreType.DMA,
        pltpu.SemaphoreType.DMA,
    ],
    compiler_params=pltpu.CompilerParams(needs_layout_passes=False),
)
def _hist_sc(x_hbm, out_hbm, buf0, buf1, table, sem0, sem1):
    wid = lax.axis_index("s") * 2 + lax.axis_index("c")
    base = wid * PER_TILE

    zeros16 = jnp.zeros((LANES,), jnp.float32)

    @plsc.parallel_loop(0, TW // LANES, unroll=4)
    def _zero(i):
        table[pl.ds(i * LANES, LANES)] = zeros16

    lane = lax.iota(jnp.int32, LANES)
    ones16 = jnp.ones((LANES,), jnp.float32)

    def process(buf):
        @plsc.parallel_loop(0, CHUNK // LANES, unroll=UNROLL)
        def _body(i):
            v = buf[pl.ds(i * LANES, LANES)]
            t = v * INV_W + BIAS
            t = jnp.minimum(jnp.maximum(t, 0.0), float(N_LEVELS + 1))
            idx = lax.shift_left(t.astype(jnp.int32), 4) + lane
            plsc.addupdate_scatter(table, [idx], ones16)

    def start(g, buf, sem):
        off = pl.multiple_of(base + g * CHUNK, CHUNK)
        return pltpu.async_copy(x_hbm.at[pl.ds(off, CHUNK)], buf, sem)

    def wait(buf, sem):
        pltpu.make_async_copy(x_hbm.at[pl.ds(base, CHUNK)], buf, sem).wait()

    start(0, buf0, sem0)
    start(1, buf1, sem1)

    def pair_body(p, c):
        g = p * 2
        wait(buf0, sem0)
        process(buf0)
        start(g + 2, buf0, sem0)
        wait(buf1, sem1)
        process(buf1)
        start(g + 3, buf1, sem1)
        return c

    lax.fori_loop(0, NPAIR - 1, pair_body, 0)

    wait(buf0, sem0)
    process(buf0)
    wait(buf1, sem1)
    process(buf1)

    @plsc.parallel_loop(0, TW // LANES, unroll=4)
    def _scale(i):
        table[pl.ds(i * LANES, LANES)] = table[pl.ds(i * LANES, LANES)] * SCALE

    pltpu.sync_copy(table, out_hbm.at[wid])


def kernel(x, bins_edges):
    parts = _hist_sc(x.reshape(TOTAL))
    density = jnp.sum(parts.reshape(NW, NBINS, LANES), axis=(0, 2))[1 : N_LEVELS + 1]
    return (x, density)


# speedup vs baseline: 3.5888x; 1.0764x over previous
import functools

import jax
import jax.numpy as jnp
from jax import lax
from jax.experimental import pallas as pl
from jax.experimental.pallas import tpu as pltpu
from jax.experimental.pallas import tpu_sc as plsc

N_LEVELS = 256
VMIN, VMAX = -4.0, 4.0
Q_STEP = (VMAX - VMIN) / (N_LEVELS - 1)
INV_W = 1.0 / Q_STEP
BIAS = -(VMIN - Q_STEP / 2.0) * INV_W + 1.0

LANES = 16
NW = 32
NBINS = 258
TW = NBINS * LANES
TOTAL = 1 * 16 * 2048 * 2048
PER_TILE = TOTAL // NW
CHUNK = 32768
NCHUNK = PER_TILE // CHUNK
NPAIR = NCHUNK // 2
UNROLL = 8
SCALE = 1.0 / TOTAL

_mesh = plsc.VectorSubcoreMesh(core_axis_name="c", subcore_axis_name="s")


@functools.partial(
    pl.kernel,
    mesh=_mesh,
    out_type=jax.ShapeDtypeStruct((NW, TW), jnp.float32),
    scratch_types=[
        pltpu.VMEM((CHUNK,), jnp.float32),
        pltpu.VMEM((CHUNK,), jnp.float32),
        pltpu.VMEM((TW,), jnp.float32),
        pltpu.SemaphoreType.DMA,
        pltpu.SemaphoreType.DMA,
    ],
    compiler_params=pltpu.CompilerParams(needs_layout_passes=False),
)
def _hist_sc(x_hbm, out_hbm, buf0, buf1, table, sem0, sem1):
    wid = lax.axis_index("s") * 2 + lax.axis_index("c")
    base = wid * PER_TILE

    zeros16 = jnp.zeros((LANES,), jnp.float32)

    @plsc.parallel_loop(0, TW // LANES, unroll=4)
    def _zero(i):
        table[pl.ds(i * LANES, LANES)] = zeros16

    lane = lax.iota(jnp.int32, LANES)
    ones16 = jnp.ones((LANES,), jnp.float32)

    def process(buf):
        @plsc.parallel_loop(0, CHUNK // LANES, unroll=UNROLL)
        def _body(i):
            v = buf[pl.ds(i * LANES, LANES)]
            t = v * INV_W + BIAS
            t = jnp.minimum(jnp.maximum(t, 0.0), float(N_LEVELS + 1))
            idx = lax.shift_left(t.astype(jnp.int32), 4) + lane
            plsc.addupdate_scatter(table, [idx], ones16)

    def start(g, buf, sem):
        off = pl.multiple_of(base + g * CHUNK, CHUNK)
        return pltpu.async_copy(x_hbm.at[pl.ds(off, CHUNK)], buf, sem)

    def wait(buf, sem):
        pltpu.make_async_copy(x_hbm.at[pl.ds(base, CHUNK)], buf, sem).wait()

    start(0, buf0, sem0)
    start(1, buf1, sem1)

    def pair_body(p, c):
        g = p * 2
        wait(buf0, sem0)
        process(buf0)
        start(g + 2, buf0, sem0)
        wait(buf1, sem1)
        process(buf1)
        start(g + 3, buf1, sem1)
        return c

    lax.fori_loop(0, NPAIR - 1, pair_body, 0)

    wait(buf0, sem0)
    process(buf0)
    wait(buf1, sem1)
    process(buf1)

    @plsc.parallel_loop(0, TW // LANES, unroll=4)
    def _scale(i):
        table[pl.ds(i * LANES, LANES)] = table[pl.ds(i * LANES, LANES)] * SCALE

    pltpu.sync_copy(table, out_hbm.at[wid])


def kernel(x, bins_edges):
    parts = _hist_sc(x.reshape(TOTAL))
    density = jnp.sum(parts.reshape(NW, NBINS, LANES), axis=(0, 2))[1 : N_LEVELS + 1]
    return (x, density)
